# baseline (speedup 1.0000x reference)
import functools

import jax
import jax.numpy as jnp
from jax.experimental import pallas as pl
from jax.experimental.pallas import tpu as pltpu

_ALPHA = 0.25
_IOU_LOW = 0.4
_IOU_HIGH = 0.5
_W_CLS = 1.0
_W_REG = 1.0

_BM1 = 2048
_BM2 = 2048
_BM3 = 4096


def _deint(v, tmat):
    rows = v.shape[0]
    p = jax.lax.dot_general(v, tmat, (((1,), (0,)), ((), ())),
                            preferred_element_type=jnp.float32)
    planes = []
    for c in range(4):
        pc = p[:, 128 * c:128 * (c + 1)]
        planes.append(jnp.concatenate(
            [pc[r:r + 1, :] for r in range(rows)], axis=1))
    return planes


def _match_body(N, anch_ref, tmat_ref, gtb_ref, mv_ref, mt_ref,
                gmax_ref, garg_ref):
    j = pl.program_id(1)
    ax0, ay0, ax1, ay1 = _deint(anch_ref[...], tmat_ref[...])
    g = gtb_ref[0]
    gx0, gy0, gx1, gy1 = g[:, 0:1], g[:, 1:2], g[:, 2:3], g[:, 3:4]
    area_a = (ax1 - ax0) * (ay1 - ay0)
    area_g = (gx1 - gx0) * (gy1 - gy0)
    w = jnp.maximum(jnp.minimum(gx1, ax1) - jnp.maximum(gx0, ax0), 0.0)
    h = jnp.maximum(jnp.minimum(gy1, ay1) - jnp.maximum(gy0, ay0), 0.0)
    inter = w * h
    iou = inter / (area_g + area_a - inter)

    mv = jnp.max(iou, axis=0, keepdims=True)
    gt_ids = jax.lax.broadcasted_iota(jnp.int32, iou.shape, 0)
    mt = jnp.min(jnp.where(iou == mv, gt_ids, N), axis=0, keepdims=True)
    mv_ref[0] = mv
    mt_ref[0] = mt

    rmax = jnp.max(iou, axis=1, keepdims=True)
    lane = jax.lax.broadcasted_iota(jnp.int32, iou.shape, 1) + j * _BM1
    rarg = jnp.min(jnp.where(iou == rmax, lane, jnp.int32(2**30)),
                   axis=1, keepdims=True)

    @pl.when(j == 0)
    def _():
        gmax_ref[0] = rmax
        garg_ref[0] = rarg

    @pl.when(j > 0)
    def _():
        cur = gmax_ref[0]
        better = rmax > cur
        gmax_ref[0] = jnp.where(better, rmax, cur)
        garg_ref[0] = jnp.where(better, rarg, garg_ref[0])


def _labels_fg(N, mv, garg, j, bm):
    labels = jnp.where(mv < _IOU_LOW, 0, jnp.where(mv < _IOU_HIGH, -1, 1))
    lane = jax.lax.broadcasted_iota(jnp.int32, (N, bm), 1) + j * bm
    lq = jnp.any(garg == lane, axis=0, keepdims=True)
    return jnp.where(lq, 1, labels)


def _assign_body(B, N, nj, mv_ref, mt_ref, garg_ref, gtl_ref, mask_ref,
                 tc_ref, valid_ref, fgc_ref, acc_ref):
    b = pl.program_id(0)
    j = pl.program_id(1)
    mv = mv_ref[0]
    mt = mt_ref[0]
    labels = _labels_fg(N, mv, garg_ref[0], j, _BM2)
    fg = labels == 1
    validf = (labels != -1).astype(jnp.float32) * mask_ref[0]

    gt_ids = jax.lax.broadcasted_iota(jnp.int32, (N, _BM2), 0)
    eq = (gt_ids == mt).astype(jnp.float32)
    glab = gtl_ref[0].astype(jnp.float32)
    tc = jnp.sum(eq * glab, axis=0, keepdims=True).astype(jnp.int32)
    tc_ref[0] = jnp.where(fg, tc, -1)
    valid_ref[0] = validf

    fg_c = jnp.sum(fg.astype(jnp.float32))
    first = (b == 0) & (j == 0)

    @pl.when(first)
    def _():
        acc_ref[0] = fg_c

    @pl.when(jnp.logical_not(first))
    def _():
        acc_ref[0] += fg_c

    @pl.when((b == B - 1) & (j == nj - 1))
    def _():
        lanes = jax.lax.broadcasted_iota(jnp.int32, (1, 128), 1)
        fgc_ref[...] = jnp.where(lanes == 0, acc_ref[0], 0.0)


def _focal_body(pc_ref, tc_ref, valid_ref, fgc_ref, out_ref, acc_ref):
    k = pl.program_id(0)
    nk = pl.num_programs(0)
    x = pc_ref[...]
    tc = tc_ref[...]
    vf = valid_ref[...]
    cls_id = jax.lax.broadcasted_iota(jnp.int32, x.shape, 1)
    t = cls_id == tc
    y = jnp.where(t, -x, x)
    e = jnp.exp(-jnp.abs(y))
    sp = jnp.maximum(y, 0.0) + jnp.log1p(e)
    sig = jnp.where(y >= 0, 1.0, e) / (1.0 + e)
    w = jnp.where(t, _ALPHA, 1.0 - _ALPHA) * vf
    contrib = jnp.sum(w * sp * (sig * sig))

    @pl.when(k == 0)
    def _():
        acc_ref[0] = contrib

    @pl.when(k > 0)
    def _():
        acc_ref[0] += contrib

    @pl.when(k == nk - 1)
    def _():
        num_fg = jnp.maximum(fgc_ref[0], 1.0)
        lanes = jax.lax.broadcasted_iota(jnp.int32, (1, 128), 1)
        out_ref[...] = jnp.where(lanes == 0, acc_ref[0] / num_fg, 0.0)


def _giou_body(B, N, nj, mv_ref, mt_ref, garg_ref, gtb_ref,
               pv_ref, tmat_ref, ll_ref, fgc_ref, out_ref, acc_ref):
    b = pl.program_id(0)
    j = pl.program_id(1)
    labels = _labels_fg(N, mv_ref[0], garg_ref[0], j, _BM2)
    fgf = (labels == 1).astype(jnp.float32)

    gt_ids = jax.lax.broadcasted_iota(jnp.int32, (N, _BM2), 0)
    eq = (gt_ids == mt_ref[0]).astype(jnp.float32)
    g = gtb_ref[0]
    tx0 = jnp.sum(eq * g[:, 0:1], axis=0, keepdims=True)
    ty0 = jnp.sum(eq * g[:, 1:2], axis=0, keepdims=True)
    tx1 = jnp.sum(eq * g[:, 2:3], axis=0, keepdims=True)
    ty1 = jnp.sum(eq * g[:, 3:4], axis=0, keepdims=True)

    px0, py0, px1, py1 = _deint(pv_ref[0], tmat_ref[...])
    a1 = (px1 - px0) * (py1 - py0)
    a2 = (tx1 - tx0) * (ty1 - ty0)
    w = jnp.maximum(jnp.minimum(px1, tx1) - jnp.maximum(px0, tx0), 0.0)
    h = jnp.maximum(jnp.minimum(py1, ty1) - jnp.maximum(py0, ty0), 0.0)
    inter = w * h
    union = a1 + a2 - inter
    iou2 = inter / union
    wc = jnp.maximum(jnp.maximum(px1, tx1) - jnp.minimum(px0, tx0), 0.0)
    hc = jnp.maximum(jnp.maximum(py1, ty1) - jnp.minimum(py0, ty0), 0.0)
    areac = wc * hc
    giou = iou2 - (areac - union) / areac

    box_c = jnp.sum((1.0 - giou) * fgf)
    first = (b == 0) & (j == 0)

    @pl.when(first)
    def _():
        acc_ref[0] = box_c

    @pl.when(jnp.logical_not(first))
    def _():
        acc_ref[0] += box_c

    @pl.when((b == B - 1) & (j == nj - 1))
    def _():
        num_fg = jnp.maximum(fgc_ref[0], 1.0)
        ll = ll_ref[0]
        lb = acc_ref[0] / num_fg
        lanes = jax.lax.broadcasted_iota(jnp.int32, (1, 128), 1)
        out_ref[...] = jnp.where(
            lanes == 0, ll,
            jnp.where(lanes == 1, lb,
                      jnp.where(lanes == 2, _W_CLS * ll + _W_REG * lb, 0.0)))


@jax.jit
def kernel(pred_cls, pred_box, mask, anchor_boxes, tgt_boxes, tgt_labels):
    B, M, C = pred_cls.shape
    N = tgt_boxes.shape[1]
    nj1 = M // _BM1
    nj2 = M // _BM2

    gtl = tgt_labels.astype(jnp.int32).reshape(B, N, 1)
    maskf = mask.astype(jnp.float32).reshape(B, 1, M)
    av = anchor_boxes.reshape(M // 128, 512)
    pv = pred_box.reshape(B, M // 128, 512)
    ki = jax.lax.broadcasted_iota(jnp.int32, (512, 512), 0)
    li = jax.lax.broadcasted_iota(jnp.int32, (512, 512), 1)
    lc, la = li // 128, li % 128
    tmat = (ki == 128 * (la // 32) + 4 * (la % 32) + lc).astype(jnp.float32)

    seq = pltpu.CompilerParams(dimension_semantics=("arbitrary", "arbitrary"))

    mv, mt, _, garg = pl.pallas_call(
        functools.partial(_match_body, N),
        grid=(B, nj1),
        in_specs=[
            pl.BlockSpec((_BM1 // 128, 512), lambda b, j: (j, 0)),
            pl.BlockSpec((512, 512), lambda b, j: (0, 0)),
            pl.BlockSpec((1, N, 4), lambda b, j: (b, 0, 0)),
        ],
        out_specs=[
            pl.BlockSpec((1, 1, _BM1), lambda b, j: (b, 0, j)),
            pl.BlockSpec((1, 1, _BM1), lambda b, j: (b, 0, j)),
            pl.BlockSpec((1, N, 1), lambda b, j: (b, 0, 0)),
            pl.BlockSpec((1, N, 1), lambda b, j: (b, 0, 0)),
        ],
        out_shape=[
            jax.ShapeDtypeStruct((B, 1, M), jnp.float32),
            jax.ShapeDtypeStruct((B, 1, M), jnp.int32),
            jax.ShapeDtypeStruct((B, N, 1), jnp.float32),
            jax.ShapeDtypeStruct((B, N, 1), jnp.int32),
        ],
        compiler_params=seq,
    )(av, tmat, tgt_boxes)

    tc_eff, validf, fgc = pl.pallas_call(
        functools.partial(_assign_body, B, N, nj2),
        grid=(B, nj2),
        in_specs=[
            pl.BlockSpec((1, 1, _BM2), lambda b, j: (b, 0, j)),
            pl.BlockSpec((1, 1, _BM2), lambda b, j: (b, 0, j)),
            pl.BlockSpec((1, N, 1), lambda b, j: (b, 0, 0)),
            pl.BlockSpec((1, N, 1), lambda b, j: (b, 0, 0)),
            pl.BlockSpec((1, 1, _BM2), lambda b, j: (b, 0, j)),
        ],
        out_specs=[
            pl.BlockSpec((1, 1, _BM2), lambda b, j: (b, 0, j)),
            pl.BlockSpec((1, 1, _BM2), lambda b, j: (b, 0, j)),
            pl.BlockSpec((1, 128), lambda b, j: (0, 0)),
        ],
        out_shape=[
            jax.ShapeDtypeStruct((B, 1, M), jnp.int32),
            jax.ShapeDtypeStruct((B, 1, M), jnp.float32),
            jax.ShapeDtypeStruct((1, 128), jnp.float32),
        ],
        scratch_shapes=[pltpu.SMEM((1,), jnp.float32)],
        compiler_params=seq,
    )(mv, mt, garg, gtl, maskf)

    pc_flat = pred_cls.reshape(B * M, C)
    tc_flat = tc_eff.reshape(B * M, 1)
    vf_flat = validf.reshape(B * M, 1)

    ll_vec = pl.pallas_call(
        _focal_body,
        grid=(B * M // _BM3,),
        in_specs=[
            pl.BlockSpec((_BM3, C), lambda k: (k, 0)),
            pl.BlockSpec((_BM3, 1), lambda k: (k, 0)),
            pl.BlockSpec((_BM3, 1), lambda k: (k, 0)),
            pl.BlockSpec(memory_space=pltpu.SMEM),
        ],
        out_specs=pl.BlockSpec((1, 128), lambda k: (0, 0)),
        out_shape=jax.ShapeDtypeStruct((1, 128), jnp.float32),
        scratch_shapes=[pltpu.SMEM((1,), jnp.float32)],
        compiler_params=pltpu.CompilerParams(
            dimension_semantics=("arbitrary",)),
    )(pc_flat, tc_flat, vf_flat, fgc[0, :1])

    out = pl.pallas_call(
        functools.partial(_giou_body, B, N, nj2),
        grid=(B, nj2),
        in_specs=[
            pl.BlockSpec((1, 1, _BM2), lambda b, j: (b, 0, j)),
            pl.BlockSpec((1, 1, _BM2), lambda b, j: (b, 0, j)),
            pl.BlockSpec((1, N, 1), lambda b, j: (b, 0, 0)),
            pl.BlockSpec((1, N, 4), lambda b, j: (b, 0, 0)),
            pl.BlockSpec((1, _BM2 // 128, 512), lambda b, j: (b, j, 0)),
            pl.BlockSpec((512, 512), lambda b, j: (0, 0)),
            pl.BlockSpec(memory_space=pltpu.SMEM),
            pl.BlockSpec(memory_space=pltpu.SMEM),
        ],
        out_specs=pl.BlockSpec((1, 128), lambda b, j: (0, 0)),
        out_shape=jax.ShapeDtypeStruct((1, 128), jnp.float32),
        scratch_shapes=[pltpu.SMEM((1,), jnp.float32)],
        compiler_params=seq,
    )(mv, mt, garg, tgt_boxes, pv, tmat, ll_vec[0, :1], fgc[0, :1])

    return out[0, 0], out[0, 1], out[0, 2]


# speedup vs baseline: 1.2796x; 1.2796x over previous
import functools

import jax
import jax.numpy as jnp
from jax.experimental import pallas as pl
from jax.experimental.pallas import tpu as pltpu

_ALPHA = 0.25
_IOU_LOW = 0.4
_IOU_HIGH = 0.5
_W_CLS = 1.0
_W_REG = 1.0

_BM1 = 2048
_BM2 = 2048
_BM3 = 4096


def _match_body(N, anch_ref, gtb_ref, mv_ref, mt_ref, gmax_ref, garg_ref):
    j = pl.program_id(1)
    a = anch_ref[...]
    ax0, ay0, ax1, ay1 = a[0:1], a[1:2], a[2:3], a[3:4]
    g = gtb_ref[0]
    gx0, gy0, gx1, gy1 = g[:, 0:1], g[:, 1:2], g[:, 2:3], g[:, 3:4]
    area_a = (ax1 - ax0) * (ay1 - ay0)
    area_g = (gx1 - gx0) * (gy1 - gy0)
    w = jnp.maximum(jnp.minimum(gx1, ax1) - jnp.maximum(gx0, ax0), 0.0)
    h = jnp.maximum(jnp.minimum(gy1, ay1) - jnp.maximum(gy0, ay0), 0.0)
    inter = w * h
    iou = inter / (area_g + area_a - inter)

    mv = jnp.max(iou, axis=0, keepdims=True)
    gt_ids = jax.lax.broadcasted_iota(jnp.int32, iou.shape, 0)
    mt = jnp.min(jnp.where(iou == mv, gt_ids, N), axis=0, keepdims=True)
    mv_ref[0] = mv
    mt_ref[0] = mt

    rmax = jnp.max(iou, axis=1, keepdims=True)
    lane = jax.lax.broadcasted_iota(jnp.int32, iou.shape, 1) + j * _BM1
    rarg = jnp.min(jnp.where(iou == rmax, lane, jnp.int32(2**30)),
                   axis=1, keepdims=True)

    @pl.when(j == 0)
    def _():
        gmax_ref[0] = rmax
        garg_ref[0] = rarg

    @pl.when(j > 0)
    def _():
        cur = gmax_ref[0]
        better = rmax > cur
        gmax_ref[0] = jnp.where(better, rmax, cur)
        garg_ref[0] = jnp.where(better, rarg, garg_ref[0])


def _assign_body(B, N, nj, mv_ref, mt_ref, garg_ref, gtl_ref, gtb_ref, pb_ref,
                 mask_ref, tc_ref, valid_ref, sums_ref, acc_ref):
    b = pl.program_id(0)
    j = pl.program_id(1)
    mv = mv_ref[0]
    mt = mt_ref[0]
    labels = jnp.where(mv < _IOU_LOW, 0, jnp.where(mv < _IOU_HIGH, -1, 1))
    garg = garg_ref[0]
    lane = jax.lax.broadcasted_iota(jnp.int32, (N, _BM2), 1) + j * _BM2
    lq = jnp.any(garg == lane, axis=0, keepdims=True)
    labels = jnp.where(lq, 1, labels)
    fg = labels == 1
    fgf = fg.astype(jnp.float32)
    validf = (labels != -1).astype(jnp.float32) * mask_ref[0]

    gt_ids = jax.lax.broadcasted_iota(jnp.int32, (N, _BM2), 0)
    eq = (gt_ids == mt).astype(jnp.float32)
    glab = gtl_ref[0].astype(jnp.float32)
    tc = jnp.sum(eq * glab, axis=0, keepdims=True).astype(jnp.int32)
    tc_ref[0] = jnp.where(fg, tc, -1)
    valid_ref[0] = validf

    g = gtb_ref[0]
    tx0 = jnp.sum(eq * g[:, 0:1], axis=0, keepdims=True)
    ty0 = jnp.sum(eq * g[:, 1:2], axis=0, keepdims=True)
    tx1 = jnp.sum(eq * g[:, 2:3], axis=0, keepdims=True)
    ty1 = jnp.sum(eq * g[:, 3:4], axis=0, keepdims=True)

    p = pb_ref[0]
    px0, py0, px1, py1 = p[0:1], p[1:2], p[2:3], p[3:4]
    a1 = (px1 - px0) * (py1 - py0)
    a2 = (tx1 - tx0) * (ty1 - ty0)
    w = jnp.maximum(jnp.minimum(px1, tx1) - jnp.maximum(px0, tx0), 0.0)
    h = jnp.maximum(jnp.minimum(py1, ty1) - jnp.maximum(py0, ty0), 0.0)
    inter = w * h
    union = a1 + a2 - inter
    iou2 = inter / union
    wc = jnp.maximum(jnp.maximum(px1, tx1) - jnp.minimum(px0, tx0), 0.0)
    hc = jnp.maximum(jnp.maximum(py1, ty1) - jnp.minimum(py0, ty0), 0.0)
    areac = wc * hc
    giou = iou2 - (areac - union) / areac

    box_c = jnp.sum((1.0 - giou) * fgf)
    fg_c = jnp.sum(fgf)
    first = (b == 0) & (j == 0)

    @pl.when(first)
    def _():
        acc_ref[0] = box_c
        acc_ref[1] = fg_c

    @pl.when(jnp.logical_not(first))
    def _():
        acc_ref[0] += box_c
        acc_ref[1] += fg_c

    @pl.when((b == B - 1) & (j == nj - 1))
    def _():
        lanes = jax.lax.broadcasted_iota(jnp.int32, (1, 128), 1)
        sums_ref[...] = jnp.where(lanes == 0, acc_ref[0],
                                  jnp.where(lanes == 1, acc_ref[1], 0.0))


def _focal_body(pc_ref, tc_ref, valid_ref, sums_ref, out_ref, acc_ref):
    k = pl.program_id(0)
    nk = pl.num_programs(0)
    x = pc_ref[...]
    tc = tc_ref[...]
    vf = valid_ref[...]
    cls_id = jax.lax.broadcasted_iota(jnp.int32, x.shape, 1)
    t = cls_id == tc
    y = jnp.where(t, -x, x)
    e = jnp.exp(-jnp.abs(y))
    sp = jnp.maximum(y, 0.0) + jnp.log1p(e)
    sig = jnp.where(y >= 0, 1.0, e) / (1.0 + e)
    w = jnp.where(t, _ALPHA, 1.0 - _ALPHA) * vf
    contrib = jnp.sum(w * sp * (sig * sig))

    @pl.when(k == 0)
    def _():
        acc_ref[0] = contrib

    @pl.when(k > 0)
    def _():
        acc_ref[0] += contrib

    @pl.when(k == nk - 1)
    def _():
        box_sum = sums_ref[0]
        fg_c = sums_ref[1]
        num_fg = jnp.maximum(fg_c, 1.0)
        ll = acc_ref[0] / num_fg
        lb = box_sum / num_fg
        lanes = jax.lax.broadcasted_iota(jnp.int32, (1, 128), 1)
        out_ref[...] = jnp.where(
            lanes == 0, ll,
            jnp.where(lanes == 1, lb,
                      jnp.where(lanes == 2, _W_CLS * ll + _W_REG * lb, 0.0)))


@jax.jit
def kernel(pred_cls, pred_box, mask, anchor_boxes, tgt_boxes, tgt_labels):
    B, M, C = pred_cls.shape
    N = tgt_boxes.shape[1]
    nj1 = M // _BM1
    nj2 = M // _BM2

    anch_t = anchor_boxes.T
    pb_t = jnp.transpose(pred_box, (0, 2, 1))
    gtl = tgt_labels.astype(jnp.int32).reshape(B, N, 1)
    maskf = mask.astype(jnp.float32).reshape(B, 1, M)

    seq = pltpu.CompilerParams(dimension_semantics=("arbitrary", "arbitrary"))

    mv, mt, _, garg = pl.pallas_call(
        functools.partial(_match_body, N),
        grid=(B, nj1),
        in_specs=[
            pl.BlockSpec((4, _BM1), lambda b, j: (0, j)),
            pl.BlockSpec((1, N, 4), lambda b, j: (b, 0, 0)),
        ],
        out_specs=[
            pl.BlockSpec((1, 1, _BM1), lambda b, j: (b, 0, j)),
            pl.BlockSpec((1, 1, _BM1), lambda b, j: (b, 0, j)),
            pl.BlockSpec((1, N, 1), lambda b, j: (b, 0, 0)),
            pl.BlockSpec((1, N, 1), lambda b, j: (b, 0, 0)),
        ],
        out_shape=[
            jax.ShapeDtypeStruct((B, 1, M), jnp.float32),
            jax.ShapeDtypeStruct((B, 1, M), jnp.int32),
            jax.ShapeDtypeStruct((B, N, 1), jnp.float32),
            jax.ShapeDtypeStruct((B, N, 1), jnp.int32),
        ],
        compiler_params=seq,
    )(anch_t, tgt_boxes)

    tc_eff, validf, sums = pl.pallas_call(
        functools.partial(_assign_body, B, N, nj2),
        grid=(B, nj2),
        in_specs=[
            pl.BlockSpec((1, 1, _BM2), lambda b, j: (b, 0, j)),
            pl.BlockSpec((1, 1, _BM2), lambda b, j: (b, 0, j)),
            pl.BlockSpec((1, N, 1), lambda b, j: (b, 0, 0)),
            pl.BlockSpec((1, N, 1), lambda b, j: (b, 0, 0)),
            pl.BlockSpec((1, N, 4), lambda b, j: (b, 0, 0)),
            pl.BlockSpec((1, 4, _BM2), lambda b, j: (b, 0, j)),
            pl.BlockSpec((1, 1, _BM2), lambda b, j: (b, 0, j)),
        ],
        out_specs=[
            pl.BlockSpec((1, 1, _BM2), lambda b, j: (b, 0, j)),
            pl.BlockSpec((1, 1, _BM2), lambda b, j: (b, 0, j)),
            pl.BlockSpec((1, 128), lambda b, j: (0, 0)),
        ],
        out_shape=[
            jax.ShapeDtypeStruct((B, 1, M), jnp.int32),
            jax.ShapeDtypeStruct((B, 1, M), jnp.float32),
            jax.ShapeDtypeStruct((1, 128), jnp.float32),
        ],
        scratch_shapes=[pltpu.SMEM((2,), jnp.float32)],
        compiler_params=seq,
    )(mv, mt, garg, gtl, tgt_boxes, pb_t, maskf)

    pc_flat = pred_cls.reshape(B * M, C)
    tc_flat = tc_eff.reshape(B * M, 1)
    vf_flat = validf.reshape(B * M, 1)

    out = pl.pallas_call(
        _focal_body,
        grid=(B * M // _BM3,),
        in_specs=[
            pl.BlockSpec((_BM3, C), lambda k: (k, 0)),
            pl.BlockSpec((_BM3, 1), lambda k: (k, 0)),
            pl.BlockSpec((_BM3, 1), lambda k: (k, 0)),
            pl.BlockSpec(memory_space=pltpu.SMEM),
        ],
        out_specs=pl.BlockSpec((1, 128), lambda k: (0, 0)),
        out_shape=jax.ShapeDtypeStruct((1, 128), jnp.float32),
        scratch_shapes=[pltpu.SMEM((1,), jnp.float32)],
        compiler_params=pltpu.CompilerParams(
            dimension_semantics=("arbitrary",)),
    )(pc_flat, tc_flat, vf_flat, sums[0, :2])

    return out[0, 0], out[0, 1], out[0, 2]


# speedup vs baseline: 1.4398x; 1.1252x over previous
import functools

import jax
import jax.numpy as jnp
from jax.experimental import pallas as pl
from jax.experimental.pallas import tpu as pltpu

_ALPHA = 0.25
_IOU_LOW = 0.4
_IOU_HIGH = 0.5
_W_CLS = 1.0
_W_REG = 1.0

_BM1 = 4096
_BM2 = 2048
_BM3 = 8192


def _match_body(N, anch_ref, gtb_ref, mv_ref, mt_ref, gmax_ref, garg_ref):
    j = pl.program_id(1)
    a = anch_ref[...]
    ax0, ay0, ax1, ay1 = a[0:1], a[1:2], a[2:3], a[3:4]
    g = gtb_ref[0]
    gx0, gy0, gx1, gy1 = g[:, 0:1], g[:, 1:2], g[:, 2:3], g[:, 3:4]
    area_a = (ax1 - ax0) * (ay1 - ay0)
    area_g = (gx1 - gx0) * (gy1 - gy0)
    w = jnp.maximum(jnp.minimum(gx1, ax1) - jnp.maximum(gx0, ax0), 0.0)
    h = jnp.maximum(jnp.minimum(gy1, ay1) - jnp.maximum(gy0, ay0), 0.0)
    inter = w * h
    iou = inter / (area_g + area_a - inter)

    mv = jnp.max(iou, axis=0, keepdims=True)
    gt_ids = jax.lax.broadcasted_iota(jnp.int32, iou.shape, 0)
    mt = jnp.min(jnp.where(iou == mv, gt_ids, N), axis=0, keepdims=True)
    mv_ref[0] = mv
    mt_ref[0] = mt

    rmax = jnp.max(iou, axis=1, keepdims=True)
    lane = jax.lax.broadcasted_iota(jnp.int32, iou.shape, 1) + j * _BM1
    rarg = jnp.min(jnp.where(iou == rmax, lane, jnp.int32(2**30)),
                   axis=1, keepdims=True)

    @pl.when(j == 0)
    def _():
        gmax_ref[0] = rmax
        garg_ref[0] = rarg

    @pl.when(j > 0)
    def _():
        cur = gmax_ref[0]
        better = rmax > cur
        gmax_ref[0] = jnp.where(better, rmax, cur)
        garg_ref[0] = jnp.where(better, rarg, garg_ref[0])


def _assign_body(B, N, nj, mv_ref, mt_ref, garg_ref, gtl_ref, gtb_ref, pb_ref,
                 mask_ref, tc_ref, valid_ref, sums_ref, acc_ref):
    b = pl.program_id(0)
    j = pl.program_id(1)
    mv = mv_ref[0]
    mt = mt_ref[0]
    labels = jnp.where(mv < _IOU_LOW, 0, jnp.where(mv < _IOU_HIGH, -1, 1))
    garg = garg_ref[0]
    lane = jax.lax.broadcasted_iota(jnp.int32, (N, _BM2), 1) + j * _BM2
    lq = jnp.any(garg == lane, axis=0, keepdims=True)
    labels = jnp.where(lq, 1, labels)
    fg = labels == 1
    fgf = fg.astype(jnp.float32)
    validf = (labels != -1).astype(jnp.float32) * mask_ref[0]

    gt_ids = jax.lax.broadcasted_iota(jnp.int32, (N, _BM2), 0)
    eq = (gt_ids == mt).astype(jnp.float32)
    glab = gtl_ref[0].astype(jnp.float32)
    tc = jnp.sum(eq * glab, axis=0, keepdims=True).astype(jnp.int32)
    tc_ref[0] = jnp.where(fg, tc, -1)
    valid_ref[0] = validf

    g = gtb_ref[0]
    tx0 = jnp.sum(eq * g[:, 0:1], axis=0, keepdims=True)
    ty0 = jnp.sum(eq * g[:, 1:2], axis=0, keepdims=True)
    tx1 = jnp.sum(eq * g[:, 2:3], axis=0, keepdims=True)
    ty1 = jnp.sum(eq * g[:, 3:4], axis=0, keepdims=True)

    p = pb_ref[0]
    px0, py0, px1, py1 = p[0:1], p[1:2], p[2:3], p[3:4]
    a1 = (px1 - px0) * (py1 - py0)
    a2 = (tx1 - tx0) * (ty1 - ty0)
    w = jnp.maximum(jnp.minimum(px1, tx1) - jnp.maximum(px0, tx0), 0.0)
    h = jnp.maximum(jnp.minimum(py1, ty1) - jnp.maximum(py0, ty0), 0.0)
    inter = w * h
    union = a1 + a2 - inter
    iou2 = inter / union
    wc = jnp.maximum(jnp.maximum(px1, tx1) - jnp.minimum(px0, tx0), 0.0)
    hc = jnp.maximum(jnp.maximum(py1, ty1) - jnp.minimum(py0, ty0), 0.0)
    areac = wc * hc
    giou = iou2 - (areac - union) / areac

    box_c = jnp.sum((1.0 - giou) * fgf)
    fg_c = jnp.sum(fgf)
    first = (b == 0) & (j == 0)

    @pl.when(first)
    def _():
        acc_ref[0] = box_c
        acc_ref[1] = fg_c

    @pl.when(jnp.logical_not(first))
    def _():
        acc_ref[0] += box_c
        acc_ref[1] += fg_c

    @pl.when((b == B - 1) & (j == nj - 1))
    def _():
        lanes = jax.lax.broadcasted_iota(jnp.int32, (1, 128), 1)
        sums_ref[...] = jnp.where(lanes == 0, acc_ref[0],
                                  jnp.where(lanes == 1, acc_ref[1], 0.0))


def _focal_body(pc_ref, tc_ref, valid_ref, sums_ref, out_ref, acc_ref):
    k = pl.program_id(0)
    nk = pl.num_programs(0)
    x = pc_ref[...]
    tc = tc_ref[...]
    vf = valid_ref[...]
    cls_id = jax.lax.broadcasted_iota(jnp.int32, x.shape, 1)
    t = cls_id == tc
    y = jnp.where(t, -x, x)
    e = jnp.exp(-jnp.abs(y))
    sp = jnp.maximum(y, 0.0) + jnp.log1p(e)
    sig = jnp.where(y >= 0, 1.0, e) / (1.0 + e)
    w = jnp.where(t, _ALPHA, 1.0 - _ALPHA) * vf
    contrib = jnp.sum(w * sp * (sig * sig))

    @pl.when(k == 0)
    def _():
        acc_ref[0] = contrib

    @pl.when(k > 0)
    def _():
        acc_ref[0] += contrib

    @pl.when(k == nk - 1)
    def _():
        box_sum = sums_ref[0]
        fg_c = sums_ref[1]
        num_fg = jnp.maximum(fg_c, 1.0)
        ll = acc_ref[0] / num_fg
        lb = box_sum / num_fg
        lanes = jax.lax.broadcasted_iota(jnp.int32, (1, 128), 1)
        out_ref[...] = jnp.where(
            lanes == 0, ll,
            jnp.where(lanes == 1, lb,
                      jnp.where(lanes == 2, _W_CLS * ll + _W_REG * lb, 0.0)))


@jax.jit
def kernel(pred_cls, pred_box, mask, anchor_boxes, tgt_boxes, tgt_labels):
    B, M, C = pred_cls.shape
    N = tgt_boxes.shape[1]
    nj1 = M // _BM1
    nj2 = M // _BM2

    anch_t = anchor_boxes.T
    pb_t = jnp.transpose(pred_box, (0, 2, 1))
    gtl = tgt_labels.astype(jnp.int32).reshape(B, N, 1)
    maskf = mask.astype(jnp.float32).reshape(B, 1, M)

    seq = pltpu.CompilerParams(dimension_semantics=("arbitrary", "arbitrary"))

    mv, mt, _, garg = pl.pallas_call(
        functools.partial(_match_body, N),
        grid=(B, nj1),
        in_specs=[
            pl.BlockSpec((4, _BM1), lambda b, j: (0, j)),
            pl.BlockSpec((1, N, 4), lambda b, j: (b, 0, 0)),
        ],
        out_specs=[
            pl.BlockSpec((1, 1, _BM1), lambda b, j: (b, 0, j)),
            pl.BlockSpec((1, 1, _BM1), lambda b, j: (b, 0, j)),
            pl.BlockSpec((1, N, 1), lambda b, j: (b, 0, 0)),
            pl.BlockSpec((1, N, 1), lambda b, j: (b, 0, 0)),
        ],
        out_shape=[
            jax.ShapeDtypeStruct((B, 1, M), jnp.float32),
            jax.ShapeDtypeStruct((B, 1, M), jnp.int32),
            jax.ShapeDtypeStruct((B, N, 1), jnp.float32),
            jax.ShapeDtypeStruct((B, N, 1), jnp.int32),
        ],
        compiler_params=seq,
    )(anch_t, tgt_boxes)

    tc_eff, validf, sums = pl.pallas_call(
        functools.partial(_assign_body, B, N, nj2),
        grid=(B, nj2),
        in_specs=[
            pl.BlockSpec((1, 1, _BM2), lambda b, j: (b, 0, j)),
            pl.BlockSpec((1, 1, _BM2), lambda b, j: (b, 0, j)),
            pl.BlockSpec((1, N, 1), lambda b, j: (b, 0, 0)),
            pl.BlockSpec((1, N, 1), lambda b, j: (b, 0, 0)),
            pl.BlockSpec((1, N, 4), lambda b, j: (b, 0, 0)),
            pl.BlockSpec((1, 4, _BM2), lambda b, j: (b, 0, j)),
            pl.BlockSpec((1, 1, _BM2), lambda b, j: (b, 0, j)),
        ],
        out_specs=[
            pl.BlockSpec((1, 1, _BM2), lambda b, j: (b, 0, j)),
            pl.BlockSpec((1, 1, _BM2), lambda b, j: (b, 0, j)),
            pl.BlockSpec((1, 128), lambda b, j: (0, 0)),
        ],
        out_shape=[
            jax.ShapeDtypeStruct((B, 1, M), jnp.int32),
            jax.ShapeDtypeStruct((B, 1, M), jnp.float32),
            jax.ShapeDtypeStruct((1, 128), jnp.float32),
        ],
        scratch_shapes=[pltpu.SMEM((2,), jnp.float32)],
        compiler_params=seq,
    )(mv, mt, garg, gtl, tgt_boxes, pb_t, maskf)

    pc_flat = pred_cls.reshape(B * M, C)
    tc_flat = tc_eff.reshape(B * M, 1)
    vf_flat = validf.reshape(B * M, 1)

    out = pl.pallas_call(
        _focal_body,
        grid=(B * M // _BM3,),
        in_specs=[
            pl.BlockSpec((_BM3, C), lambda k: (k, 0)),
            pl.BlockSpec((_BM3, 1), lambda k: (k, 0)),
            pl.BlockSpec((_BM3, 1), lambda k: (k, 0)),
            pl.BlockSpec(memory_space=pltpu.SMEM),
        ],
        out_specs=pl.BlockSpec((1, 128), lambda k: (0, 0)),
        out_shape=jax.ShapeDtypeStruct((1, 128), jnp.float32),
        scratch_shapes=[pltpu.SMEM((1,), jnp.float32)],
        compiler_params=pltpu.CompilerParams(
            dimension_semantics=("arbitrary",)),
    )(pc_flat, tc_flat, vf_flat, sums[0, :2])

    return out[0, 0], out[0, 1], out[0, 2]


# speedup vs baseline: 1.5454x; 1.0733x over previous
import functools

import jax
import jax.numpy as jnp
from jax.experimental import pallas as pl
from jax.experimental.pallas import tpu as pltpu

_ALPHA = 0.25
_IOU_LOW = 0.4
_IOU_HIGH = 0.5
_W_CLS = 1.0
_W_REG = 1.0

_BM1 = 4096
_BM2 = 4096
_BM3 = 8192


def _match_body(N, anch_ref, gtb_ref, mv_ref, mt_ref, gmax_ref, garg_ref):
    j = pl.program_id(1)
    a = anch_ref[...]
    ax0, ay0, ax1, ay1 = a[0:1], a[1:2], a[2:3], a[3:4]
    g = gtb_ref[0]
    gx0, gy0, gx1, gy1 = g[:, 0:1], g[:, 1:2], g[:, 2:3], g[:, 3:4]
    area_a = (ax1 - ax0) * (ay1 - ay0)
    area_g = (gx1 - gx0) * (gy1 - gy0)
    w = jnp.maximum(jnp.minimum(gx1, ax1) - jnp.maximum(gx0, ax0), 0.0)
    h = jnp.maximum(jnp.minimum(gy1, ay1) - jnp.maximum(gy0, ay0), 0.0)
    inter = w * h
    iou = inter / (area_g + area_a - inter)

    mv = jnp.max(iou, axis=0, keepdims=True)
    gt_ids = jax.lax.broadcasted_iota(jnp.int32, iou.shape, 0)
    mt = jnp.min(jnp.where(iou == mv, gt_ids, N), axis=0, keepdims=True)
    mv_ref[0] = mv
    mt_ref[0] = mt

    rmax = jnp.max(iou, axis=1, keepdims=True)
    lane = jax.lax.broadcasted_iota(jnp.int32, iou.shape, 1) + j * _BM1
    rarg = jnp.min(jnp.where(iou == rmax, lane, jnp.int32(2**30)),
                   axis=1, keepdims=True)

    @pl.when(j == 0)
    def _():
        gmax_ref[0] = rmax
        garg_ref[0] = rarg

    @pl.when(j > 0)
    def _():
        cur = gmax_ref[0]
        better = rmax > cur
        gmax_ref[0] = jnp.where(better, rmax, cur)
        garg_ref[0] = jnp.where(better, rarg, garg_ref[0])


def _assign_body(B, N, nj, mv_ref, mt_ref, garg_ref, gtl_ref, gtb_ref, pb_ref,
                 mask_ref, tc_ref, valid_ref, sums_ref, acc_ref):
    b = pl.program_id(0)
    j = pl.program_id(1)
    mv = mv_ref[0]
    mt = mt_ref[0]
    labels = jnp.where(mv < _IOU_LOW, 0, jnp.where(mv < _IOU_HIGH, -1, 1))
    garg = garg_ref[0]
    lane = jax.lax.broadcasted_iota(jnp.int32, (N, _BM2), 1) + j * _BM2
    lq = jnp.any(garg == lane, axis=0, keepdims=True)
    labels = jnp.where(lq, 1, labels)
    fg = labels == 1
    fgf = fg.astype(jnp.float32)
    validf = (labels != -1).astype(jnp.float32) * mask_ref[0]

    gt_ids = jax.lax.broadcasted_iota(jnp.int32, (N, _BM2), 0)
    eq = (gt_ids == mt).astype(jnp.float32)
    glab = gtl_ref[0].astype(jnp.float32)
    tc = jnp.sum(eq * glab, axis=0, keepdims=True).astype(jnp.int32)
    tc_ref[0] = jnp.where(fg, tc, -1)
    valid_ref[0] = validf

    g = gtb_ref[0]
    tx0 = jnp.sum(eq * g[:, 0:1], axis=0, keepdims=True)
    ty0 = jnp.sum(eq * g[:, 1:2], axis=0, keepdims=True)
    tx1 = jnp.sum(eq * g[:, 2:3], axis=0, keepdims=True)
    ty1 = jnp.sum(eq * g[:, 3:4], axis=0, keepdims=True)

    p = pb_ref[0]
    px0, py0, px1, py1 = p[0:1], p[1:2], p[2:3], p[3:4]
    a1 = (px1 - px0) * (py1 - py0)
    a2 = (tx1 - tx0) * (ty1 - ty0)
    w = jnp.maximum(jnp.minimum(px1, tx1) - jnp.maximum(px0, tx0), 0.0)
    h = jnp.maximum(jnp.minimum(py1, ty1) - jnp.maximum(py0, ty0), 0.0)
    inter = w * h
    union = a1 + a2 - inter
    iou2 = inter / union
    wc = jnp.maximum(jnp.maximum(px1, tx1) - jnp.minimum(px0, tx0), 0.0)
    hc = jnp.maximum(jnp.maximum(py1, ty1) - jnp.minimum(py0, ty0), 0.0)
    areac = wc * hc
    giou = iou2 - (areac - union) / areac

    box_c = jnp.sum((1.0 - giou) * fgf)
    fg_c = jnp.sum(fgf)
    first = (b == 0) & (j == 0)

    @pl.when(first)
    def _():
        acc_ref[0] = box_c
        acc_ref[1] = fg_c

    @pl.when(jnp.logical_not(first))
    def _():
        acc_ref[0] += box_c
        acc_ref[1] += fg_c

    @pl.when((b == B - 1) & (j == nj - 1))
    def _():
        lanes = jax.lax.broadcasted_iota(jnp.int32, (1, 128), 1)
        sums_ref[...] = jnp.where(lanes == 0, acc_ref[0],
                                  jnp.where(lanes == 1, acc_ref[1], 0.0))


def _focal_body(pc_ref, tc_ref, valid_ref, sums_ref, out_ref, acc_ref):
    k = pl.program_id(0)
    nk = pl.num_programs(0)
    x = pc_ref[...]
    tc = tc_ref[...]
    vf = valid_ref[...]
    cls_id = jax.lax.broadcasted_iota(jnp.int32, x.shape, 1)
    t = cls_id == tc
    y = jnp.where(t, -x, x)
    e = jnp.exp(-jnp.abs(y))
    sp = jnp.maximum(y, 0.0) + jnp.log1p(e)
    sig = jnp.where(y >= 0, 1.0, e) / (1.0 + e)
    w = jnp.where(t, _ALPHA, 1.0 - _ALPHA) * vf
    contrib = jnp.sum(w * sp * (sig * sig))

    @pl.when(k == 0)
    def _():
        acc_ref[0] = contrib

    @pl.when(k > 0)
    def _():
        acc_ref[0] += contrib

    @pl.when(k == nk - 1)
    def _():
        box_sum = sums_ref[0]
        fg_c = sums_ref[1]
        num_fg = jnp.maximum(fg_c, 1.0)
        ll = acc_ref[0] / num_fg
        lb = box_sum / num_fg
        lanes = jax.lax.broadcasted_iota(jnp.int32, (1, 128), 1)
        out_ref[...] = jnp.where(
            lanes == 0, ll,
            jnp.where(lanes == 1, lb,
                      jnp.where(lanes == 2, _W_CLS * ll + _W_REG * lb, 0.0)))


@jax.jit
def kernel(pred_cls, pred_box, mask, anchor_boxes, tgt_boxes, tgt_labels):
    B, M, C = pred_cls.shape
    N = tgt_boxes.shape[1]
    nj1 = M // _BM1
    nj2 = M // _BM2

    anch_t = anchor_boxes.T
    pb_t = jnp.transpose(pred_box, (0, 2, 1))
    gtl = tgt_labels.astype(jnp.int32).reshape(B, N, 1)
    maskf = mask.astype(jnp.float32).reshape(B, 1, M)

    seq = pltpu.CompilerParams(dimension_semantics=("arbitrary", "arbitrary"))

    mv, mt, _, garg = pl.pallas_call(
        functools.partial(_match_body, N),
        grid=(B, nj1),
        in_specs=[
            pl.BlockSpec((4, _BM1), lambda b, j: (0, j)),
            pl.BlockSpec((1, N, 4), lambda b, j: (b, 0, 0)),
        ],
        out_specs=[
            pl.BlockSpec((1, 1, _BM1), lambda b, j: (b, 0, j)),
            pl.BlockSpec((1, 1, _BM1), lambda b, j: (b, 0, j)),
            pl.BlockSpec((1, N, 1), lambda b, j: (b, 0, 0)),
            pl.BlockSpec((1, N, 1), lambda b, j: (b, 0, 0)),
        ],
        out_shape=[
            jax.ShapeDtypeStruct((B, 1, M), jnp.float32),
            jax.ShapeDtypeStruct((B, 1, M), jnp.int32),
            jax.ShapeDtypeStruct((B, N, 1), jnp.float32),
            jax.ShapeDtypeStruct((B, N, 1), jnp.int32),
        ],
        compiler_params=seq,
    )(anch_t, tgt_boxes)

    tc_eff, validf, sums = pl.pallas_call(
        functools.partial(_assign_body, B, N, nj2),
        grid=(B, nj2),
        in_specs=[
            pl.BlockSpec((1, 1, _BM2), lambda b, j: (b, 0, j)),
            pl.BlockSpec((1, 1, _BM2), lambda b, j: (b, 0, j)),
            pl.BlockSpec((1, N, 1), lambda b, j: (b, 0, 0)),
            pl.BlockSpec((1, N, 1), lambda b, j: (b, 0, 0)),
            pl.BlockSpec((1, N, 4), lambda b, j: (b, 0, 0)),
            pl.BlockSpec((1, 4, _BM2), lambda b, j: (b, 0, j)),
            pl.BlockSpec((1, 1, _BM2), lambda b, j: (b, 0, j)),
        ],
        out_specs=[
            pl.BlockSpec((1, 1, _BM2), lambda b, j: (b, 0, j)),
            pl.BlockSpec((1, 1, _BM2), lambda b, j: (b, 0, j)),
            pl.BlockSpec((1, 128), lambda b, j: (0, 0)),
        ],
        out_shape=[
            jax.ShapeDtypeStruct((B, 1, M), jnp.int32),
            jax.ShapeDtypeStruct((B, 1, M), jnp.float32),
            jax.ShapeDtypeStruct((1, 128), jnp.float32),
        ],
        scratch_shapes=[pltpu.SMEM((2,), jnp.float32)],
        compiler_params=seq,
    )(mv, mt, garg, gtl, tgt_boxes, pb_t, maskf)

    pc_flat = pred_cls.reshape(B * M, C)
    tc_flat = tc_eff.reshape(B * M, 1)
    vf_flat = validf.reshape(B * M, 1)

    out = pl.pallas_call(
        _focal_body,
        grid=(B * M // _BM3,),
        in_specs=[
            pl.BlockSpec((_BM3, C), lambda k: (k, 0)),
            pl.BlockSpec((_BM3, 1), lambda k: (k, 0)),
            pl.BlockSpec((_BM3, 1), lambda k: (k, 0)),
            pl.BlockSpec(memory_space=pltpu.SMEM),
        ],
        out_specs=pl.BlockSpec((1, 128), lambda k: (0, 0)),
        out_shape=jax.ShapeDtypeStruct((1, 128), jnp.float32),
        scratch_shapes=[pltpu.SMEM((1,), jnp.float32)],
        compiler_params=pltpu.CompilerParams(
            dimension_semantics=("arbitrary",)),
    )(pc_flat, tc_flat, vf_flat, sums[0, :2])

    return out[0, 0], out[0, 1], out[0, 2]


# speedup vs baseline: 1.6712x; 1.0814x over previous
import functools

import jax
import jax.numpy as jnp
from jax.experimental import pallas as pl
from jax.experimental.pallas import tpu as pltpu

_ALPHA = 0.25
_IOU_LOW = 0.4
_IOU_HIGH = 0.5
_W_CLS = 1.0
_W_REG = 1.0

_BM1 = 8192
_BM2 = 8192
_BM3 = 8192


def _match_body(N, anch_ref, gtb_ref, mv_ref, mt_ref, gmax_ref, garg_ref):
    j = pl.program_id(1)
    a = anch_ref[...]
    ax0, ay0, ax1, ay1 = a[0:1], a[1:2], a[2:3], a[3:4]
    g = gtb_ref[0]
    gx0, gy0, gx1, gy1 = g[:, 0:1], g[:, 1:2], g[:, 2:3], g[:, 3:4]
    area_a = (ax1 - ax0) * (ay1 - ay0)
    area_g = (gx1 - gx0) * (gy1 - gy0)
    w = jnp.maximum(jnp.minimum(gx1, ax1) - jnp.maximum(gx0, ax0), 0.0)
    h = jnp.maximum(jnp.minimum(gy1, ay1) - jnp.maximum(gy0, ay0), 0.0)
    inter = w * h
    iou = inter / (area_g + area_a - inter)

    mv = jnp.max(iou, axis=0, keepdims=True)
    gt_ids = jax.lax.broadcasted_iota(jnp.int32, iou.shape, 0)
    mt = jnp.min(jnp.where(iou == mv, gt_ids, N), axis=0, keepdims=True)
    mv_ref[0] = mv
    mt_ref[0] = mt

    rmax = jnp.max(iou, axis=1, keepdims=True)
    lane = jax.lax.broadcasted_iota(jnp.int32, iou.shape, 1) + j * _BM1
    rarg = jnp.min(jnp.where(iou == rmax, lane, jnp.int32(2**30)),
                   axis=1, keepdims=True)

    @pl.when(j == 0)
    def _():
        gmax_ref[0] = rmax
        garg_ref[0] = rarg

    @pl.when(j > 0)
    def _():
        cur = gmax_ref[0]
        better = rmax > cur
        gmax_ref[0] = jnp.where(better, rmax, cur)
        garg_ref[0] = jnp.where(better, rarg, garg_ref[0])


def _assign_body(B, N, nj, mv_ref, mt_ref, garg_ref, gtl_ref, gtb_ref, pb_ref,
                 mask_ref, tc_ref, valid_ref, sums_ref, acc_ref):
    b = pl.program_id(0)
    j = pl.program_id(1)
    mv = mv_ref[0]
    mt = mt_ref[0]
    labels = jnp.where(mv < _IOU_LOW, 0, jnp.where(mv < _IOU_HIGH, -1, 1))
    garg = garg_ref[0]
    lane = jax.lax.broadcasted_iota(jnp.int32, (N, _BM2), 1) + j * _BM2
    lq = jnp.any(garg == lane, axis=0, keepdims=True)
    labels = jnp.where(lq, 1, labels)
    fg = labels == 1
    fgf = fg.astype(jnp.float32)
    validf = (labels != -1).astype(jnp.float32) * mask_ref[0]

    gt_ids = jax.lax.broadcasted_iota(jnp.int32, (N, _BM2), 0)
    eq = (gt_ids == mt).astype(jnp.float32)
    glab = gtl_ref[0].astype(jnp.float32)
    tc = jnp.sum(eq * glab, axis=0, keepdims=True).astype(jnp.int32)
    tc_ref[0] = jnp.where(fg, tc, -1)
    valid_ref[0] = validf

    g = gtb_ref[0]
    tx0 = jnp.sum(eq * g[:, 0:1], axis=0, keepdims=True)
    ty0 = jnp.sum(eq * g[:, 1:2], axis=0, keepdims=True)
    tx1 = jnp.sum(eq * g[:, 2:3], axis=0, keepdims=True)
    ty1 = jnp.sum(eq * g[:, 3:4], axis=0, keepdims=True)

    p = pb_ref[0]
    px0, py0, px1, py1 = p[0:1], p[1:2], p[2:3], p[3:4]
    a1 = (px1 - px0) * (py1 - py0)
    a2 = (tx1 - tx0) * (ty1 - ty0)
    w = jnp.maximum(jnp.minimum(px1, tx1) - jnp.maximum(px0, tx0), 0.0)
    h = jnp.maximum(jnp.minimum(py1, ty1) - jnp.maximum(py0, ty0), 0.0)
    inter = w * h
    union = a1 + a2 - inter
    iou2 = inter / union
    wc = jnp.maximum(jnp.maximum(px1, tx1) - jnp.minimum(px0, tx0), 0.0)
    hc = jnp.maximum(jnp.maximum(py1, ty1) - jnp.minimum(py0, ty0), 0.0)
    areac = wc * hc
    giou = iou2 - (areac - union) / areac

    box_c = jnp.sum((1.0 - giou) * fgf)
    fg_c = jnp.sum(fgf)
    first = (b == 0) & (j == 0)

    @pl.when(first)
    def _():
        acc_ref[0] = box_c
        acc_ref[1] = fg_c

    @pl.when(jnp.logical_not(first))
    def _():
        acc_ref[0] += box_c
        acc_ref[1] += fg_c

    @pl.when((b == B - 1) & (j == nj - 1))
    def _():
        lanes = jax.lax.broadcasted_iota(jnp.int32, (1, 128), 1)
        sums_ref[...] = jnp.where(lanes == 0, acc_ref[0],
                                  jnp.where(lanes == 1, acc_ref[1], 0.0))


def _focal_body(pc_ref, tc_ref, valid_ref, sums_ref, out_ref, acc_ref):
    k = pl.program_id(0)
    nk = pl.num_programs(0)
    x = pc_ref[...]
    tc = tc_ref[...]
    vf = valid_ref[...]
    cls_id = jax.lax.broadcasted_iota(jnp.int32, x.shape, 1)
    t = cls_id == tc
    y = jnp.where(t, -x, x)
    e = jnp.exp(-jnp.abs(y))
    sp = jnp.maximum(y, 0.0) + jnp.log1p(e)
    sig = jnp.where(y >= 0, 1.0, e) / (1.0 + e)
    w = jnp.where(t, _ALPHA, 1.0 - _ALPHA) * vf
    contrib = jnp.sum(w * sp * (sig * sig))

    @pl.when(k == 0)
    def _():
        acc_ref[0] = contrib

    @pl.when(k > 0)
    def _():
        acc_ref[0] += contrib

    @pl.when(k == nk - 1)
    def _():
        box_sum = sums_ref[0]
        fg_c = sums_ref[1]
        num_fg = jnp.maximum(fg_c, 1.0)
        ll = acc_ref[0] / num_fg
        lb = box_sum / num_fg
        lanes = jax.lax.broadcasted_iota(jnp.int32, (1, 128), 1)
        out_ref[...] = jnp.where(
            lanes == 0, ll,
            jnp.where(lanes == 1, lb,
                      jnp.where(lanes == 2, _W_CLS * ll + _W_REG * lb, 0.0)))


@jax.jit
def kernel(pred_cls, pred_box, mask, anchor_boxes, tgt_boxes, tgt_labels):
    B, M, C = pred_cls.shape
    N = tgt_boxes.shape[1]
    nj1 = M // _BM1
    nj2 = M // _BM2

    anch_t = anchor_boxes.T
    pb_t = jnp.transpose(pred_box, (0, 2, 1))
    gtl = tgt_labels.astype(jnp.int32).reshape(B, N, 1)
    maskf = mask.astype(jnp.float32).reshape(B, 1, M)

    seq = pltpu.CompilerParams(dimension_semantics=("arbitrary", "arbitrary"))

    mv, mt, _, garg = pl.pallas_call(
        functools.partial(_match_body, N),
        grid=(B, nj1),
        in_specs=[
            pl.BlockSpec((4, _BM1), lambda b, j: (0, j)),
            pl.BlockSpec((1, N, 4), lambda b, j: (b, 0, 0)),
        ],
        out_specs=[
            pl.BlockSpec((1, 1, _BM1), lambda b, j: (b, 0, j)),
            pl.BlockSpec((1, 1, _BM1), lambda b, j: (b, 0, j)),
            pl.BlockSpec((1, N, 1), lambda b, j: (b, 0, 0)),
            pl.BlockSpec((1, N, 1), lambda b, j: (b, 0, 0)),
        ],
        out_shape=[
            jax.ShapeDtypeStruct((B, 1, M), jnp.float32),
            jax.ShapeDtypeStruct((B, 1, M), jnp.int32),
            jax.ShapeDtypeStruct((B, N, 1), jnp.float32),
            jax.ShapeDtypeStruct((B, N, 1), jnp.int32),
        ],
        compiler_params=seq,
    )(anch_t, tgt_boxes)

    tc_eff, validf, sums = pl.pallas_call(
        functools.partial(_assign_body, B, N, nj2),
        grid=(B, nj2),
        in_specs=[
            pl.BlockSpec((1, 1, _BM2), lambda b, j: (b, 0, j)),
            pl.BlockSpec((1, 1, _BM2), lambda b, j: (b, 0, j)),
            pl.BlockSpec((1, N, 1), lambda b, j: (b, 0, 0)),
            pl.BlockSpec((1, N, 1), lambda b, j: (b, 0, 0)),
            pl.BlockSpec((1, N, 4), lambda b, j: (b, 0, 0)),
            pl.BlockSpec((1, 4, _BM2), lambda b, j: (b, 0, j)),
            pl.BlockSpec((1, 1, _BM2), lambda b, j: (b, 0, j)),
        ],
        out_specs=[
            pl.BlockSpec((1, 1, _BM2), lambda b, j: (b, 0, j)),
            pl.BlockSpec((1, 1, _BM2), lambda b, j: (b, 0, j)),
            pl.BlockSpec((1, 128), lambda b, j: (0, 0)),
        ],
        out_shape=[
            jax.ShapeDtypeStruct((B, 1, M), jnp.int32),
            jax.ShapeDtypeStruct((B, 1, M), jnp.float32),
            jax.ShapeDtypeStruct((1, 128), jnp.float32),
        ],
        scratch_shapes=[pltpu.SMEM((2,), jnp.float32)],
        compiler_params=seq,
    )(mv, mt, garg, gtl, tgt_boxes, pb_t, maskf)

    pc_flat = pred_cls.reshape(B * M, C)
    tc_flat = tc_eff.reshape(B * M, 1)
    vf_flat = validf.reshape(B * M, 1)

    out = pl.pallas_call(
        _focal_body,
        grid=(B * M // _BM3,),
        in_specs=[
            pl.BlockSpec((_BM3, C), lambda k: (k, 0)),
            pl.BlockSpec((_BM3, 1), lambda k: (k, 0)),
            pl.BlockSpec((_BM3, 1), lambda k: (k, 0)),
            pl.BlockSpec(memory_space=pltpu.SMEM),
        ],
        out_specs=pl.BlockSpec((1, 128), lambda k: (0, 0)),
        out_shape=jax.ShapeDtypeStruct((1, 128), jnp.float32),
        scratch_shapes=[pltpu.SMEM((1,), jnp.float32)],
        compiler_params=pltpu.CompilerParams(
            dimension_semantics=("arbitrary",)),
    )(pc_flat, tc_flat, vf_flat, sums[0, :2])

    return out[0, 0], out[0, 1], out[0, 2]


# speedup vs baseline: 1.6959x; 1.0148x over previous
import functools

import jax
import jax.numpy as jnp
from jax.experimental import pallas as pl
from jax.experimental.pallas import tpu as pltpu

_ALPHA = 0.25
_IOU_LOW = 0.4
_IOU_HIGH = 0.5
_W_CLS = 1.0
_W_REG = 1.0

_BM1 = 16384
_BM2 = 16384
_BM3 = 8192


def _match_body(N, anch_ref, gtb_ref, mv_ref, mt_ref, gmax_ref, garg_ref):
    j = pl.program_id(1)
    a = anch_ref[...]
    ax0, ay0, ax1, ay1 = a[0:1], a[1:2], a[2:3], a[3:4]
    g = gtb_ref[0]
    gx0, gy0, gx1, gy1 = g[:, 0:1], g[:, 1:2], g[:, 2:3], g[:, 3:4]
    area_a = (ax1 - ax0) * (ay1 - ay0)
    area_g = (gx1 - gx0) * (gy1 - gy0)
    w = jnp.maximum(jnp.minimum(gx1, ax1) - jnp.maximum(gx0, ax0), 0.0)
    h = jnp.maximum(jnp.minimum(gy1, ay1) - jnp.maximum(gy0, ay0), 0.0)
    inter = w * h
    iou = inter / (area_g + area_a - inter)

    mv = jnp.max(iou, axis=0, keepdims=True)
    gt_ids = jax.lax.broadcasted_iota(jnp.int32, iou.shape, 0)
    mt = jnp.min(jnp.where(iou == mv, gt_ids, N), axis=0, keepdims=True)
    mv_ref[0] = mv
    mt_ref[0] = mt

    rmax = jnp.max(iou, axis=1, keepdims=True)
    lane = jax.lax.broadcasted_iota(jnp.int32, iou.shape, 1) + j * _BM1
    rarg = jnp.min(jnp.where(iou == rmax, lane, jnp.int32(2**30)),
                   axis=1, keepdims=True)

    @pl.when(j == 0)
    def _():
        gmax_ref[0] = rmax
        garg_ref[0] = rarg

    @pl.when(j > 0)
    def _():
        cur = gmax_ref[0]
        better = rmax > cur
        gmax_ref[0] = jnp.where(better, rmax, cur)
        garg_ref[0] = jnp.where(better, rarg, garg_ref[0])


def _assign_body(B, N, nj, mv_ref, mt_ref, garg_ref, gtl_ref, gtb_ref, pb_ref,
                 mask_ref, tc_ref, valid_ref, sums_ref, acc_ref):
    b = pl.program_id(0)
    j = pl.program_id(1)
    mv = mv_ref[0]
    mt = mt_ref[0]
    labels = jnp.where(mv < _IOU_LOW, 0, jnp.where(mv < _IOU_HIGH, -1, 1))
    garg = garg_ref[0]
    lane = jax.lax.broadcasted_iota(jnp.int32, (N, _BM2), 1) + j * _BM2
    lq = jnp.any(garg == lane, axis=0, keepdims=True)
    labels = jnp.where(lq, 1, labels)
    fg = labels == 1
    fgf = fg.astype(jnp.float32)
    validf = (labels != -1).astype(jnp.float32) * mask_ref[0]

    gt_ids = jax.lax.broadcasted_iota(jnp.int32, (N, _BM2), 0)
    eq = (gt_ids == mt).astype(jnp.float32)
    glab = gtl_ref[0].astype(jnp.float32)
    tc = jnp.sum(eq * glab, axis=0, keepdims=True).astype(jnp.int32)
    tc_ref[0] = jnp.where(fg, tc, -1)
    valid_ref[0] = validf

    g = gtb_ref[0]
    tx0 = jnp.sum(eq * g[:, 0:1], axis=0, keepdims=True)
    ty0 = jnp.sum(eq * g[:, 1:2], axis=0, keepdims=True)
    tx1 = jnp.sum(eq * g[:, 2:3], axis=0, keepdims=True)
    ty1 = jnp.sum(eq * g[:, 3:4], axis=0, keepdims=True)

    p = pb_ref[0]
    px0, py0, px1, py1 = p[0:1], p[1:2], p[2:3], p[3:4]
    a1 = (px1 - px0) * (py1 - py0)
    a2 = (tx1 - tx0) * (ty1 - ty0)
    w = jnp.maximum(jnp.minimum(px1, tx1) - jnp.maximum(px0, tx0), 0.0)
    h = jnp.maximum(jnp.minimum(py1, ty1) - jnp.maximum(py0, ty0), 0.0)
    inter = w * h
    union = a1 + a2 - inter
    iou2 = inter / union
    wc = jnp.maximum(jnp.maximum(px1, tx1) - jnp.minimum(px0, tx0), 0.0)
    hc = jnp.maximum(jnp.maximum(py1, ty1) - jnp.minimum(py0, ty0), 0.0)
    areac = wc * hc
    giou = iou2 - (areac - union) / areac

    box_c = jnp.sum((1.0 - giou) * fgf)
    fg_c = jnp.sum(fgf)
    first = (b == 0) & (j == 0)

    @pl.when(first)
    def _():
        acc_ref[0] = box_c
        acc_ref[1] = fg_c

    @pl.when(jnp.logical_not(first))
    def _():
        acc_ref[0] += box_c
        acc_ref[1] += fg_c

    @pl.when((b == B - 1) & (j == nj - 1))
    def _():
        lanes = jax.lax.broadcasted_iota(jnp.int32, (1, 128), 1)
        sums_ref[...] = jnp.where(lanes == 0, acc_ref[0],
                                  jnp.where(lanes == 1, acc_ref[1], 0.0))


def _focal_body(pc_ref, tc_ref, valid_ref, sums_ref, out_ref, acc_ref):
    k = pl.program_id(0)
    nk = pl.num_programs(0)
    x = pc_ref[...]
    tc = tc_ref[...]
    vf = valid_ref[...]
    cls_id = jax.lax.broadcasted_iota(jnp.int32, x.shape, 1)
    t = cls_id == tc
    y = jnp.where(t, -x, x)
    e = jnp.exp(-jnp.abs(y))
    sp = jnp.maximum(y, 0.0) + jnp.log1p(e)
    sig = jnp.where(y >= 0, 1.0, e) / (1.0 + e)
    w = jnp.where(t, _ALPHA, 1.0 - _ALPHA) * vf
    contrib = jnp.sum(w * sp * (sig * sig))

    @pl.when(k == 0)
    def _():
        acc_ref[0] = contrib

    @pl.when(k > 0)
    def _():
        acc_ref[0] += contrib

    @pl.when(k == nk - 1)
    def _():
        box_sum = sums_ref[0]
        fg_c = sums_ref[1]
        num_fg = jnp.maximum(fg_c, 1.0)
        ll = acc_ref[0] / num_fg
        lb = box_sum / num_fg
        lanes = jax.lax.broadcasted_iota(jnp.int32, (1, 128), 1)
        out_ref[...] = jnp.where(
            lanes == 0, ll,
            jnp.where(lanes == 1, lb,
                      jnp.where(lanes == 2, _W_CLS * ll + _W_REG * lb, 0.0)))


@jax.jit
def kernel(pred_cls, pred_box, mask, anchor_boxes, tgt_boxes, tgt_labels):
    B, M, C = pred_cls.shape
    N = tgt_boxes.shape[1]
    nj1 = M // _BM1
    nj2 = M // _BM2

    anch_t = anchor_boxes.T
    pb_t = jnp.transpose(pred_box, (0, 2, 1))
    gtl = tgt_labels.astype(jnp.int32).reshape(B, N, 1)
    maskf = mask.astype(jnp.float32).reshape(B, 1, M)

    seq = pltpu.CompilerParams(dimension_semantics=("arbitrary", "arbitrary"))

    mv, mt, _, garg = pl.pallas_call(
        functools.partial(_match_body, N),
        grid=(B, nj1),
        in_specs=[
            pl.BlockSpec((4, _BM1), lambda b, j: (0, j)),
            pl.BlockSpec((1, N, 4), lambda b, j: (b, 0, 0)),
        ],
        out_specs=[
            pl.BlockSpec((1, 1, _BM1), lambda b, j: (b, 0, j)),
            pl.BlockSpec((1, 1, _BM1), lambda b, j: (b, 0, j)),
            pl.BlockSpec((1, N, 1), lambda b, j: (b, 0, 0)),
            pl.BlockSpec((1, N, 1), lambda b, j: (b, 0, 0)),
        ],
        out_shape=[
            jax.ShapeDtypeStruct((B, 1, M), jnp.float32),
            jax.ShapeDtypeStruct((B, 1, M), jnp.int32),
            jax.ShapeDtypeStruct((B, N, 1), jnp.float32),
            jax.ShapeDtypeStruct((B, N, 1), jnp.int32),
        ],
        compiler_params=seq,
    )(anch_t, tgt_boxes)

    tc_eff, validf, sums = pl.pallas_call(
        functools.partial(_assign_body, B, N, nj2),
        grid=(B, nj2),
        in_specs=[
            pl.BlockSpec((1, 1, _BM2), lambda b, j: (b, 0, j)),
            pl.BlockSpec((1, 1, _BM2), lambda b, j: (b, 0, j)),
            pl.BlockSpec((1, N, 1), lambda b, j: (b, 0, 0)),
            pl.BlockSpec((1, N, 1), lambda b, j: (b, 0, 0)),
            pl.BlockSpec((1, N, 4), lambda b, j: (b, 0, 0)),
            pl.BlockSpec((1, 4, _BM2), lambda b, j: (b, 0, j)),
            pl.BlockSpec((1, 1, _BM2), lambda b, j: (b, 0, j)),
        ],
        out_specs=[
            pl.BlockSpec((1, 1, _BM2), lambda b, j: (b, 0, j)),
            pl.BlockSpec((1, 1, _BM2), lambda b, j: (b, 0, j)),
            pl.BlockSpec((1, 128), lambda b, j: (0, 0)),
        ],
        out_shape=[
            jax.ShapeDtypeStruct((B, 1, M), jnp.int32),
            jax.ShapeDtypeStruct((B, 1, M), jnp.float32),
            jax.ShapeDtypeStruct((1, 128), jnp.float32),
        ],
        scratch_shapes=[pltpu.SMEM((2,), jnp.float32)],
        compiler_params=seq,
    )(mv, mt, garg, gtl, tgt_boxes, pb_t, maskf)

    pc_flat = pred_cls.reshape(B * M, C)
    tc_flat = tc_eff.reshape(B * M, 1)
    vf_flat = validf.reshape(B * M, 1)

    out = pl.pallas_call(
        _focal_body,
        grid=(B * M // _BM3,),
        in_specs=[
            pl.BlockSpec((_BM3, C), lambda k: (k, 0)),
            pl.BlockSpec((_BM3, 1), lambda k: (k, 0)),
            pl.BlockSpec((_BM3, 1), lambda k: (k, 0)),
            pl.BlockSpec(memory_space=pltpu.SMEM),
        ],
        out_specs=pl.BlockSpec((1, 128), lambda k: (0, 0)),
        out_shape=jax.ShapeDtypeStruct((1, 128), jnp.float32),
        scratch_shapes=[pltpu.SMEM((1,), jnp.float32)],
        compiler_params=pltpu.CompilerParams(
            dimension_semantics=("arbitrary",)),
    )(pc_flat, tc_flat, vf_flat, sums[0, :2])

    return out[0, 0], out[0, 1], out[0, 2]
